# baseline (speedup 1.0000x reference)
import functools
import math

import jax
import jax.numpy as jnp
from jax import lax
from jax.experimental import pallas as pl
from jax.experimental.pallas import tpu as pltpu
from jax.experimental.pallas import tpu_sc as plsc

VOCAB = 100000
EMBED_DIM = 64
LANES = 16

IBLK = 128
NBUF = 4


def _position_encoding(max_len, d_model):
    position = jnp.arange(max_len, dtype=jnp.float32)[:, None]
    div_term = jnp.exp(
        jnp.arange(0, d_model, 2, dtype=jnp.float32) * (-math.log(10000.0) / d_model)
    )
    pe = jnp.zeros((max_len, d_model), dtype=jnp.float32)
    pe = pe.at[:, 0::2].set(jnp.sin(position * div_term))
    pe = pe.at[:, 1::2].set(jnp.cos(position * div_term))
    return pe


def kernel(x, table):
    B, S = x.shape
    V, D = table.shape
    assert D == EMBED_DIM and B % IBLK == 0

    info = plsc.get_sparse_core_info()
    NC, NS = info.num_cores, info.num_subcores
    NW = NC * NS

    IT = B // IBLK
    n_units = S * IT
    assert n_units % (NW * NBUF) == 0
    per_worker_units = n_units // NW
    n_groups = per_worker_units // NBUF
    per_worker_tok = per_worker_units * IBLK

    UNIT = D * IBLK
    RUN = 8 * IBLK

    xT_flat = jnp.swapaxes(x, 0, 1).reshape(S * B).astype(jnp.int32)
    pe = _position_encoding(S, D)

    mesh = plsc.VectorSubcoreMesh(core_axis_name="c", subcore_axis_name="s")

    @functools.partial(
        pl.kernel,
        mesh=mesh,
        compiler_params=pltpu.CompilerParams(
            use_tc_tiling_on_sc=False, needs_layout_passes=False),
        out_type=jax.ShapeDtypeStruct((S * B * D,), jnp.float32),
        scratch_types=[
            pltpu.VMEM((per_worker_tok,), jnp.int32),
            pltpu.VMEM((S, D), jnp.float32),
            pltpu.VMEM((NBUF, IBLK, D), jnp.float32),
            pltpu.VMEM((NBUF, UNIT), jnp.float32),
            pltpu.SemaphoreType.DMA((NBUF,)),
            pltpu.SemaphoreType.DMA((NBUF,)),
        ],
    )
    def _sc_kernel(x_hbm, pe_hbm, table_hbm, out_hbm,
                   idx_v, pe_v, buf_g, buf_t, gsem, osem):
        wid = lax.axis_index("s") * NC + lax.axis_index("c")
        unit0 = wid * per_worker_units

        pltpu.sync_copy(pe_hbm, pe_v)
        pltpu.sync_copy(x_hbm.at[pl.ds(unit0 * IBLK, per_worker_tok)], idx_v)

        iota = lax.iota(jnp.int32, LANES)
        st_off = [(iota + c4 * LANES) * IBLK for c4 in range(D // LANES)]

        def start_gather(lu, b):
            pltpu.async_copy(
                table_hbm.at[idx_v.at[pl.ds(lu * IBLK, IBLK)]],
                buf_g.at[b], gsem.at[b])

        def store_dmas(b, u):
            j = u // IT
            it = lax.rem(u, IT)
            base = (j * 8 * IT + it) * RUN
            return [
                pltpu.make_async_copy(
                    buf_t.at[b, pl.ds(dh * RUN, RUN)],
                    out_hbm.at[pl.ds(base + dh * IT * RUN, RUN)],
                    osem.at[b])
                for dh in range(8)
            ]

        for b in range(NBUF):
            start_gather(b, b)

        def group_body(g, _):
            for b in range(NBUF):
                lu = g * NBUF + b
                u = unit0 + lu
                j = u // IT
                pltpu.make_async_copy(
                    table_hbm.at[idx_v.at[pl.ds(lu * IBLK, IBLK)]],
                    buf_g.at[b], gsem.at[b]).wait()

                @pl.when(g > 0)
                def _():
                    for dma in store_dmas(b, unit0 + (g - 1) * NBUF + b):
                        dma.wait()

                pe_vecs = [pe_v[j, pl.ds(c4 * LANES, LANES)]
                           for c4 in range(D // LANES)]

                def row_body(i, _):
                    for c4 in range(D // LANES):
                        v = buf_g.at[b][i, pl.ds(c4 * LANES, LANES)] + pe_vecs[c4]
                        plsc.store_scatter(buf_t.at[b], [st_off[c4] + i], v)
                    return 0

                lax.fori_loop(0, IBLK, row_body, 0, unroll=2)

                @pl.when(lu + NBUF < per_worker_units)
                def _():
                    start_gather(lu + NBUF, b)

                for dma in store_dmas(b, u):
                    dma.start()
            return 0

        lax.fori_loop(0, n_groups, group_body, 0)
        for b in range(NBUF):
            for dma in store_dmas(b, unit0 + (n_groups - 1) * NBUF + b):
                dma.wait()

    out_flat = _sc_kernel(xT_flat, pe, table)
    out5 = out_flat.reshape(S, D // 8, IT, 8, IBLK)
    return out5.transpose(2, 4, 0, 1, 3).reshape(B, S, D)


# speedup vs baseline: 1.5264x; 1.5264x over previous
import functools
import math

import jax
import jax.numpy as jnp
from jax import lax
from jax.experimental import pallas as pl
from jax.experimental.pallas import tpu as pltpu
from jax.experimental.pallas import tpu_sc as plsc

VOCAB = 100000
EMBED_DIM = 64
LANES = 16

IBLK = 128
NBUF = 4


def _position_encoding(max_len, d_model):
    position = jnp.arange(max_len, dtype=jnp.float32)[:, None]
    div_term = jnp.exp(
        jnp.arange(0, d_model, 2, dtype=jnp.float32) * (-math.log(10000.0) / d_model)
    )
    pe = jnp.zeros((max_len, d_model), dtype=jnp.float32)
    pe = pe.at[:, 0::2].set(jnp.sin(position * div_term))
    pe = pe.at[:, 1::2].set(jnp.cos(position * div_term))
    return pe


def kernel(x, table):
    B, S = x.shape
    V, D = table.shape
    assert D == EMBED_DIM and B % IBLK == 0

    info = plsc.get_sparse_core_info()
    NC, NS = info.num_cores, info.num_subcores
    NW = NC * NS

    IT = B // IBLK
    n_units = S * IT
    assert n_units % (NW * NBUF) == 0
    per_worker_units = n_units // NW
    n_groups = per_worker_units // NBUF
    per_worker_tok = per_worker_units * IBLK

    UNIT = D * IBLK
    RUN = 8 * IBLK

    xT_flat = jnp.swapaxes(x, 0, 1).reshape(S * B).astype(jnp.int32)
    pe = _position_encoding(S, D)

    mesh = plsc.VectorSubcoreMesh(core_axis_name="c", subcore_axis_name="s")

    @functools.partial(
        pl.kernel,
        mesh=mesh,
        compiler_params=pltpu.CompilerParams(
            use_tc_tiling_on_sc=False, needs_layout_passes=False),
        out_type=jax.ShapeDtypeStruct((S * B * D,), jnp.float32),
        scratch_types=[
            pltpu.VMEM((per_worker_tok,), jnp.int32),
            pltpu.VMEM((S, D), jnp.float32),
            pltpu.VMEM((NBUF, IBLK, D), jnp.float32),
            pltpu.VMEM((NBUF, UNIT), jnp.float32),
            pltpu.SemaphoreType.DMA((NBUF,)),
            pltpu.SemaphoreType.DMA((NBUF,)),
        ],
    )
    def _sc_kernel(x_hbm, pe_hbm, table_hbm, out_hbm,
                   idx_v, pe_v, buf_g, buf_t, gsem, osem):
        wid = lax.axis_index("s") * NC + lax.axis_index("c")
        unit0 = wid * per_worker_units

        pltpu.sync_copy(pe_hbm, pe_v)
        pltpu.sync_copy(x_hbm.at[pl.ds(unit0 * IBLK, per_worker_tok)], idx_v)

        iota = lax.iota(jnp.int32, LANES)
        st_off = [(iota + c4 * LANES) * IBLK for c4 in range(D // LANES)]

        def start_gather(lu, b):
            pltpu.async_copy(
                table_hbm.at[idx_v.at[pl.ds(lu * IBLK, IBLK)]],
                buf_g.at[b], gsem.at[b])

        def store_dmas(b, u):
            j = u // IT
            it = lax.rem(u, IT)
            base = (j * 8 * IT + it) * RUN
            return [
                pltpu.make_async_copy(
                    buf_t.at[b, pl.ds(dh * RUN, RUN)],
                    out_hbm.at[pl.ds(base + dh * IT * RUN, RUN)],
                    osem.at[b])
                for dh in range(8)
            ]

        for b in range(NBUF):
            start_gather(b, b)

        def group_body(g, _):
            for b in range(NBUF):
                lu = g * NBUF + b
                u = unit0 + lu
                j = u // IT
                pltpu.make_async_copy(
                    table_hbm.at[idx_v.at[pl.ds(lu * IBLK, IBLK)]],
                    buf_g.at[b], gsem.at[b]).wait()

                @pl.when(g > 0)
                def _():
                    for dma in store_dmas(b, unit0 + (g - 1) * NBUF + b):
                        dma.wait()

                pe_vecs = [pe_v[j, pl.ds(c4 * LANES, LANES)]
                           for c4 in range(D // LANES)]

                @plsc.parallel_loop(0, IBLK, unroll=4)
                def _(i):
                    for c4 in range(D // LANES):
                        v = buf_g.at[b][i, pl.ds(c4 * LANES, LANES)] + pe_vecs[c4]
                        plsc.store_scatter(buf_t.at[b], [st_off[c4] + i], v)

                @pl.when(lu + NBUF < per_worker_units)
                def _():
                    start_gather(lu + NBUF, b)

                for dma in store_dmas(b, u):
                    dma.start()
            return 0

        lax.fori_loop(0, n_groups, group_body, 0)
        for b in range(NBUF):
            for dma in store_dmas(b, unit0 + (n_groups - 1) * NBUF + b):
                dma.wait()

    out_flat = _sc_kernel(xT_flat, pe, table)
    out5 = out_flat.reshape(S, D // 8, IT, 8, IBLK)
    return out5.transpose(2, 4, 0, 1, 3).reshape(B, S, D)


# speedup vs baseline: 1.5268x; 1.0003x over previous
import functools
import math

import jax
import jax.numpy as jnp
from jax import lax
from jax.experimental import pallas as pl
from jax.experimental.pallas import tpu as pltpu
from jax.experimental.pallas import tpu_sc as plsc

VOCAB = 100000
EMBED_DIM = 64
LANES = 16

IBLK = 128
NBUF = 4


def _position_encoding(max_len, d_model):
    position = jnp.arange(max_len, dtype=jnp.float32)[:, None]
    div_term = jnp.exp(
        jnp.arange(0, d_model, 2, dtype=jnp.float32) * (-math.log(10000.0) / d_model)
    )
    pe = jnp.zeros((max_len, d_model), dtype=jnp.float32)
    pe = pe.at[:, 0::2].set(jnp.sin(position * div_term))
    pe = pe.at[:, 1::2].set(jnp.cos(position * div_term))
    return pe


def kernel(x, table):
    B, S = x.shape
    V, D = table.shape
    assert D == EMBED_DIM and B % IBLK == 0

    info = plsc.get_sparse_core_info()
    NC, NS = info.num_cores, info.num_subcores
    NW = NC * NS

    IT = B // IBLK
    n_units = S * IT
    assert n_units % (NW * NBUF) == 0
    per_worker_units = n_units // NW
    n_groups = per_worker_units // NBUF
    per_worker_tok = per_worker_units * IBLK

    UNIT = D * IBLK
    RUN = 8 * IBLK

    xT_flat = jnp.swapaxes(x, 0, 1).reshape(S * B).astype(jnp.int32)
    pe = _position_encoding(S, D)

    mesh = plsc.VectorSubcoreMesh(core_axis_name="c", subcore_axis_name="s")

    @functools.partial(
        pl.kernel,
        mesh=mesh,
        compiler_params=pltpu.CompilerParams(
            use_tc_tiling_on_sc=False, needs_layout_passes=False),
        out_type=jax.ShapeDtypeStruct((S * B * D,), jnp.float32),
        scratch_types=[
            pltpu.VMEM((per_worker_tok,), jnp.int32),
            pltpu.VMEM((S, D), jnp.float32),
            pltpu.VMEM((NBUF, IBLK, D), jnp.float32),
            pltpu.VMEM((NBUF, UNIT), jnp.float32),
            pltpu.SemaphoreType.DMA((NBUF,)),
            pltpu.SemaphoreType.DMA((NBUF,)),
        ],
    )
    def _sc_kernel(x_hbm, pe_hbm, table_hbm, out_hbm,
                   idx_v, pe_v, buf_g, buf_t, gsem, osem):
        wid = lax.axis_index("s") * NC + lax.axis_index("c")
        unit0 = wid * per_worker_units

        pltpu.sync_copy(pe_hbm, pe_v)
        pltpu.sync_copy(x_hbm.at[pl.ds(unit0 * IBLK, per_worker_tok)], idx_v)

        iota = lax.iota(jnp.int32, LANES)
        st_off = [(iota + c4 * LANES) * IBLK for c4 in range(D // LANES)]

        def start_gather(lu, b):
            pltpu.async_copy(
                table_hbm.at[idx_v.at[pl.ds(lu * IBLK, IBLK)]],
                buf_g.at[b], gsem.at[b])

        def store_dmas(b, u):
            j = u // IT
            it = lax.rem(u, IT)
            base = (j * 8 * IT + it) * RUN
            return [
                pltpu.make_async_copy(
                    buf_t.at[b, pl.ds(dh * RUN, RUN)],
                    out_hbm.at[pl.ds(base + dh * IT * RUN, RUN)],
                    osem.at[b])
                for dh in range(8)
            ]

        for b in range(NBUF):
            start_gather(b, b)

        def group_body(g, _):
            for b in range(NBUF):
                lu = g * NBUF + b
                u = unit0 + lu
                j = u // IT
                pltpu.make_async_copy(
                    table_hbm.at[idx_v.at[pl.ds(lu * IBLK, IBLK)]],
                    buf_g.at[b], gsem.at[b]).wait()

                @pl.when(g > 0)
                def _():
                    for dma in store_dmas(b, unit0 + (g - 1) * NBUF + b):
                        dma.wait()

                pe_vecs = [pe_v[j, pl.ds(c4 * LANES, LANES)]
                           for c4 in range(D // LANES)]

                @plsc.parallel_loop(0, IBLK, unroll=8)
                def _(i):
                    for c4 in range(D // LANES):
                        v = buf_g.at[b][i, pl.ds(c4 * LANES, LANES)] + pe_vecs[c4]
                        plsc.store_scatter(buf_t.at[b], [st_off[c4] + i], v)

                @pl.when(lu + NBUF < per_worker_units)
                def _():
                    start_gather(lu + NBUF, b)

                for dma in store_dmas(b, u):
                    dma.start()
            return 0

        lax.fori_loop(0, n_groups, group_body, 0)
        for b in range(NBUF):
            for dma in store_dmas(b, unit0 + (n_groups - 1) * NBUF + b):
                dma.wait()

    out_flat = _sc_kernel(xT_flat, pe, table)
    out5 = out_flat.reshape(S, D // 8, IT, 8, IBLK)
    return out5.transpose(2, 4, 0, 1, 3).reshape(B, S, D)
